# baseline (speedup 1.0000x reference)
import functools

import jax
import jax.numpy as jnp
from jax import lax
from jax.experimental import pallas as pl
from jax.experimental.pallas import tpu as pltpu
from jax.experimental.pallas import tpu_sc as plsc

_NC = 2
_NS = 16
_NW = _NC * _NS
_CHUNK = 128
_NBUF = 4
_NSTAGE = 2
_LANES = 16
_RUNROLL = 8


def kernel(x, table):
    B, H = x.shape
    V, D = table.shape
    nb = B // _CHUNK
    idx = x.T.reshape(H, nb, _CHUNK).transpose(1, 0, 2).astype(jnp.int32)
    table_wide = jnp.pad(table, ((0, 0), (0, 128 - D)))
    tj = D // 8

    mesh = plsc.VectorSubcoreMesh(core_axis_name="c", subcore_axis_name="s")

    @functools.partial(
        pl.kernel,
        mesh=mesh,
        compiler_params=pltpu.CompilerParams(needs_layout_passes=False),
        out_type=jax.ShapeDtypeStruct((H * tj * nb, 8, 128), jnp.float32),
        scratch_types=[
            pltpu.VMEM((H, _CHUNK), jnp.int32),
            *[pltpu.VMEM((_CHUNK, 128), jnp.float32) for _ in range(_NBUF)],
            *[pltpu.VMEM((tj, 8, 128), jnp.float32) for _ in range(_NSTAGE)],
            *[pltpu.SemaphoreType.DMA for _ in range(_NBUF + _NSTAGE)],
        ],
    )
    def _embed(idx_hbm, table_hbm, out_hbm, idx_v, *bufs_and_sems):
        rows = bufs_and_sems[:_NBUF]
        stage = bufs_and_sems[_NBUF:_NBUF + _NSTAGE]
        gsem = bufs_and_sems[_NBUF + _NSTAGE:2 * _NBUF + _NSTAGE]
        ssem = bufs_and_sems[2 * _NBUF + _NSTAGE:]
        wid = lax.axis_index("s") * _NC + lax.axis_index("c")
        pltpu.sync_copy(idx_hbm.at[wid], idx_v)

        lane = lax.iota(jnp.int32, 16)
        tj_vecs = [(lane + 16 * k) >> 3 for k in range(D // _LANES)]
        sj_vecs = [(lane + 16 * k) & 7 for k in range(D // _LANES)]

        def gather_copy(h, b):
            return pltpu.make_async_copy(
                table_hbm.at[idx_v.at[h]], rows[b], gsem[b]
            )

        def store_tile(h, s, t):
            return pltpu.make_async_copy(
                stage[s].at[pl.ds(t, 1)],
                out_hbm.at[pl.ds(h * (tj * nb) + t * nb + wid, 1)],
                ssem[s],
            )

        def transpose(b, s):
            def row_body(r0, carry):
                for dr in range(_RUNROLL):
                    r = r0 * _RUNROLL + dr
                    rvec = lane * 0 + r
                    for k in range(D // _LANES):
                        plsc.store_scatter(
                            stage[s],
                            [tj_vecs[k], sj_vecs[k], rvec],
                            rows[b][r, pl.ds(k * _LANES, _LANES)],
                        )
                return carry
            lax.fori_loop(0, _CHUNK // _RUNROLL, row_body, 0)

        def swait(s):
            for t in range(tj):
                pltpu.make_async_copy(
                    stage[s].at[pl.ds(t, 1)],
                    out_hbm.at[pl.ds(t * nb, 1)],
                    ssem[s],
                ).wait()

        def process(h, b, s, do_swait, do_prefetch):
            gather_copy(h, b).wait()
            if do_swait:
                swait(s)
            transpose(b, s)
            for t in range(tj):
                store_tile(h, s, t).start()
            if do_prefetch:
                gather_copy(h + _NBUF, b).start()

        for b in range(_NBUF):
            gather_copy(b, b).start()
        for b in range(_NBUF):
            process(b, b, b % _NSTAGE, b >= _NSTAGE, True)

        def body(g, carry):
            for b in range(_NBUF):
                process(g * _NBUF + b, b, b % _NSTAGE, True, True)
            return carry

        lax.fori_loop(1, H // _NBUF - 1, body, 0)

        for b in range(_NBUF):
            process(H - _NBUF + b, b, b % _NSTAGE, True, False)
        for s in range(_NSTAGE):
            swait(s)

    out3 = _embed(idx, table_wide)
    out = (
        out3.reshape(H, tj, nb, 8, 128)
        .transpose(2, 4, 0, 1, 3)
        .reshape(B, H, D)
    )
    return out


# speedup vs baseline: 1.0046x; 1.0046x over previous
import functools

import jax
import jax.numpy as jnp
from jax import lax
from jax.experimental import pallas as pl
from jax.experimental.pallas import tpu as pltpu
from jax.experimental.pallas import tpu_sc as plsc

_NC = 2
_NS = 16
_NW = _NC * _NS
_CHUNK = 128
_NBUF = 4
_NSTAGE = 2
_LANES = 16
_RUNROLL = 8


def kernel(x, table):
    B, H = x.shape
    V, D = table.shape
    nb = B // _CHUNK
    idx = x.T.reshape(H, nb, _CHUNK).transpose(1, 0, 2).astype(jnp.int32)
    table_wide = jnp.pad(table, ((0, 0), (0, 128 - D)))
    tj = D // 8
    tile_w = 8 * 128
    unit_w = tj * tile_w

    mesh = plsc.VectorSubcoreMesh(core_axis_name="c", subcore_axis_name="s")

    @functools.partial(
        pl.kernel,
        mesh=mesh,
        compiler_params=pltpu.CompilerParams(needs_layout_passes=False),
        out_type=jax.ShapeDtypeStruct((H * tj * nb * tile_w,), jnp.float32),
        scratch_types=[
            pltpu.VMEM((H, _CHUNK), jnp.int32),
            *[pltpu.VMEM((_CHUNK, 128), jnp.float32) for _ in range(_NBUF)],
            *[pltpu.VMEM((unit_w,), jnp.float32) for _ in range(_NSTAGE)],
            *[pltpu.SemaphoreType.DMA for _ in range(_NBUF + _NSTAGE)],
        ],
    )
    def _embed(idx_hbm, table_hbm, out_hbm, idx_v, *bufs_and_sems):
        rows = bufs_and_sems[:_NBUF]
        stage = bufs_and_sems[_NBUF:_NBUF + _NSTAGE]
        gsem = bufs_and_sems[_NBUF + _NSTAGE:2 * _NBUF + _NSTAGE]
        ssem = bufs_and_sems[2 * _NBUF + _NSTAGE:]
        wid = lax.axis_index("s") * _NC + lax.axis_index("c")
        pltpu.sync_copy(idx_hbm.at[wid], idx_v)

        lane = lax.iota(jnp.int32, 16)
        joff = [
            ((lane + 16 * k) >> 3) * 1024 + ((lane + 16 * k) & 7) * 128
            for k in range(D // _LANES)
        ]

        def gather_copy(h, b):
            return pltpu.make_async_copy(
                table_hbm.at[idx_v.at[h]], rows[b], gsem[b]
            )

        def store_unit(h, s):
            for t in range(tj):
                pltpu.make_async_copy(
                    stage[s].at[pl.ds(t * tile_w, tile_w)],
                    out_hbm.at[pl.ds(((h * tj + t) * nb + wid) * tile_w,
                                     tile_w)],
                    ssem[s],
                ).start()

        def drain_unit(s):
            pltpu.make_async_copy(
                stage[s],
                out_hbm.at[pl.ds(0, unit_w)],
                ssem[s],
            ).wait()

        def transpose(b, s):
            def row_body(r0, carry):
                for dr in range(_RUNROLL):
                    r = r0 * _RUNROLL + dr
                    rvec = jnp.broadcast_to(r, (16,)).astype(jnp.int32)
                    for k in range(D // _LANES):
                        plsc.store_scatter(
                            stage[s],
                            [joff[k] + rvec],
                            rows[b][r, pl.ds(k * _LANES, _LANES)],
                        )
                return carry
            lax.fori_loop(0, _CHUNK // _RUNROLL, row_body, 0)

        def process(h, b, s, do_swait, do_prefetch):
            gather_copy(h, b).wait()
            if do_swait:
                drain_unit(s)
            transpose(b, s)
            store_unit(h, s)
            if do_prefetch:
                gather_copy(h + _NBUF, b).start()

        for b in range(_NBUF):
            gather_copy(b, b).start()
        for b in range(_NBUF):
            process(b, b, b % _NSTAGE, b >= _NSTAGE, True)

        def body(g, carry):
            for b in range(_NBUF):
                process(g * _NBUF + b, b, b % _NSTAGE, True, True)
            return carry

        lax.fori_loop(1, H // _NBUF - 1, body, 0)

        for b in range(_NBUF):
            process(H - _NBUF + b, b, b % _NSTAGE, True, False)
        for s in range(_NSTAGE):
            drain_unit(s)

    out1 = _embed(idx, table_wide)
    out = (
        out1.reshape(H, tj, nb, 8, 128)
        .transpose(2, 4, 0, 1, 3)
        .reshape(B, H, D)
    )
    return out


# speedup vs baseline: 1.5266x; 1.5196x over previous
import functools

import jax
import jax.numpy as jnp
from jax import lax
from jax.experimental import pallas as pl
from jax.experimental.pallas import tpu as pltpu
from jax.experimental.pallas import tpu_sc as plsc

_NC = 2
_NS = 16
_NW = _NC * _NS
_CHUNK = 128
_NBUF = 4
_NSTAGE = 2
_LANES = 16
_RUNROLL = 4


def kernel(x, table):
    B, H = x.shape
    V, D = table.shape
    n = B * H
    per_w = n // _NW
    nchunk = per_w // _CHUNK
    idx = x.reshape(_NW, nchunk, _CHUNK).astype(jnp.int32)
    table_wide = jnp.pad(table, ((0, 0), (0, 128 - D)))

    mesh = plsc.VectorSubcoreMesh(core_axis_name="c", subcore_axis_name="s")

    @functools.partial(
        pl.kernel,
        mesh=mesh,
        out_type=jax.ShapeDtypeStruct((n, D), jnp.float32),
        scratch_types=[
            pltpu.VMEM((nchunk, _CHUNK), jnp.int32),
            *[pltpu.VMEM((_CHUNK, 128), jnp.float32) for _ in range(_NBUF)],
            *[pltpu.VMEM((_CHUNK, D), jnp.float32) for _ in range(_NSTAGE)],
            *[pltpu.SemaphoreType.DMA for _ in range(_NBUF + _NSTAGE)],
        ],
    )
    def _embed(idx_hbm, table_hbm, out_hbm, idx_v, *bufs_and_sems):
        rows = bufs_and_sems[:_NBUF]
        stage = bufs_and_sems[_NBUF:_NBUF + _NSTAGE]
        gsem = bufs_and_sems[_NBUF + _NSTAGE:2 * _NBUF + _NSTAGE]
        ssem = bufs_and_sems[2 * _NBUF + _NSTAGE:]
        wid = lax.axis_index("s") * _NC + lax.axis_index("c")
        base = wid * per_w
        pltpu.sync_copy(idx_hbm.at[wid], idx_v)

        def gather_copy(c, b):
            return pltpu.make_async_copy(
                table_hbm.at[idx_v.at[c]], rows[b], gsem[b]
            )

        def store_copy(c, s):
            return pltpu.make_async_copy(
                stage[s],
                out_hbm.at[pl.ds(base + c * _CHUNK, _CHUNK)],
                ssem[s],
            )

        def compact(b, s):
            def row_body(r0, carry):
                for dr in range(_RUNROLL):
                    r = r0 * _RUNROLL + dr
                    for k in range(D // _LANES):
                        stage[s][r, pl.ds(k * _LANES, _LANES)] = (
                            rows[b][r, pl.ds(k * _LANES, _LANES)]
                        )
                return carry
            lax.fori_loop(0, _CHUNK // _RUNROLL, row_body, 0)

        def process(c, b, s, do_swait, do_prefetch):
            gather_copy(c, b).wait()
            if do_swait:
                store_copy(c - _NSTAGE, s).wait()
            compact(b, s)
            store_copy(c, s).start()
            if do_prefetch:
                gather_copy(c + _NBUF, b).start()

        for b in range(_NBUF):
            gather_copy(b, b).start()
        for b in range(_NBUF):
            process(b, b, b % _NSTAGE, b >= _NSTAGE, True)

        def body(g, carry):
            for b in range(_NBUF):
                process(g * _NBUF + b, b, b % _NSTAGE, True, True)
            return carry

        lax.fori_loop(1, nchunk // _NBUF - 1, body, 0)

        for b in range(_NBUF):
            process(nchunk - _NBUF + b, b, b % _NSTAGE, True, False)
        for s in range(_NSTAGE):
            store_copy(nchunk - _NSTAGE + s, s).wait()

    out = _embed(idx, table_wide)
    return out.reshape(B, H, D)


# speedup vs baseline: 1.5274x; 1.0005x over previous
import functools

import jax
import jax.numpy as jnp
from jax import lax
from jax.experimental import pallas as pl
from jax.experimental.pallas import tpu as pltpu
from jax.experimental.pallas import tpu_sc as plsc

_NC = 2
_NS = 16
_NW = _NC * _NS
_CHUNK = 128
_NBUF = 4
_NSTAGE = 2
_LANES = 16
_RUNROLL = 8


def kernel(x, table):
    B, H = x.shape
    V, D = table.shape
    n = B * H
    per_w = n // _NW
    nchunk = per_w // _CHUNK
    idx = x.reshape(_NW, nchunk, _CHUNK).astype(jnp.int32)
    table_wide = jnp.pad(table, ((0, 0), (0, 128 - D)))

    mesh = plsc.VectorSubcoreMesh(core_axis_name="c", subcore_axis_name="s")

    @functools.partial(
        pl.kernel,
        mesh=mesh,
        out_type=jax.ShapeDtypeStruct((n, D), jnp.float32),
        scratch_types=[
            pltpu.VMEM((nchunk, _CHUNK), jnp.int32),
            *[pltpu.VMEM((_CHUNK, 128), jnp.float32) for _ in range(_NBUF)],
            *[pltpu.VMEM((_CHUNK, D), jnp.float32) for _ in range(_NSTAGE)],
            *[pltpu.SemaphoreType.DMA for _ in range(_NBUF + _NSTAGE)],
        ],
    )
    def _embed(idx_hbm, table_hbm, out_hbm, idx_v, *bufs_and_sems):
        rows = bufs_and_sems[:_NBUF]
        stage = bufs_and_sems[_NBUF:_NBUF + _NSTAGE]
        gsem = bufs_and_sems[_NBUF + _NSTAGE:2 * _NBUF + _NSTAGE]
        ssem = bufs_and_sems[2 * _NBUF + _NSTAGE:]
        wid = lax.axis_index("s") * _NC + lax.axis_index("c")
        base = wid * per_w
        pltpu.sync_copy(idx_hbm.at[wid], idx_v)

        def gather_copy(c, b):
            return pltpu.make_async_copy(
                table_hbm.at[idx_v.at[c]], rows[b], gsem[b]
            )

        def store_copy(c, s):
            return pltpu.make_async_copy(
                stage[s],
                out_hbm.at[pl.ds(base + c * _CHUNK, _CHUNK)],
                ssem[s],
            )

        def compact(b, s):
            def row_body(r0, carry):
                for dr in range(_RUNROLL):
                    r = r0 * _RUNROLL + dr
                    for k in range(D // _LANES):
                        stage[s][r, pl.ds(k * _LANES, _LANES)] = (
                            rows[b][r, pl.ds(k * _LANES, _LANES)]
                        )
                return carry
            lax.fori_loop(0, _CHUNK // _RUNROLL, row_body, 0)

        def process(c, b, s, do_swait, do_prefetch):
            gather_copy(c, b).wait()
            if do_swait:
                store_copy(c - _NSTAGE, s).wait()
            compact(b, s)
            store_copy(c, s).start()
            if do_prefetch:
                gather_copy(c + _NBUF, b).start()

        for b in range(_NBUF):
            gather_copy(b, b).start()
        for b in range(_NBUF):
            process(b, b, b % _NSTAGE, b >= _NSTAGE, True)

        def body(g, carry):
            for b in range(_NBUF):
                process(g * _NBUF + b, b, b % _NSTAGE, True, True)
            return carry

        lax.fori_loop(1, nchunk // _NBUF - 1, body, 0)

        for b in range(_NBUF):
            process(nchunk - _NBUF + b, b, b % _NSTAGE, True, False)
        for s in range(_NSTAGE):
            store_copy(nchunk - _NSTAGE + s, s).wait()

    out = _embed(idx, table_wide)
    return out.reshape(B, H, D)
